# baseline (speedup 1.0000x reference)
import functools

import jax
import jax.numpy as jnp
from jax import lax
from jax.experimental import pallas as pl
from jax.experimental.pallas import tpu as pltpu
from jax.experimental.pallas import tpu_sc as plsc

D_M = 64
SCALE = 8.0
NC = 2
NS = 16
NW = NC * NS
LANES = 16


def _make_gather(B):
    assert B % NW == 0
    bpw = B // NW
    C = 512
    assert bpw % C == 0
    nchunk = bpw // C

    mesh = plsc.VectorSubcoreMesh(core_axis_name="c", subcore_axis_name="s")

    @functools.partial(
        pl.kernel,
        mesh=mesh,
        out_type=jax.ShapeDtypeStruct((B, D_M), jnp.float32),
        scratch_types=[
            pltpu.VMEM((bpw,), jnp.int32),
            pltpu.VMEM((C, D_M), jnp.float32),
            pltpu.SemaphoreType.DMA,
        ],
        compiler_params=pltpu.CompilerParams(use_tc_tiling_on_sc=False),
    )
    def k(table_hbm, idx_hbm, out_hbm, idx_v, rows_v, sem):
        wid = lax.axis_index("s") * NC + lax.axis_index("c")
        base = wid * bpw
        pltpu.sync_copy(idx_hbm.at[pl.ds(base, bpw)], idx_v)

        def chunk_body(g, carry):
            pltpu.async_copy(
                table_hbm.at[idx_v.at[pl.ds(g * C, C)]], rows_v, sem
            ).wait()

            def scale_row(r, c):
                for j in range(D_M // LANES):
                    sl = pl.ds(j * LANES, LANES)
                    rows_v[r, sl] = rows_v[r, sl] * SCALE
                return c

            lax.fori_loop(0, C, scale_row, 0)
            pltpu.sync_copy(rows_v, out_hbm.at[pl.ds(base + g * C, C)])
            return carry

        lax.fori_loop(0, nchunk, chunk_body, 0)

    return k


def kernel(x, table):
    s0, s1 = x.shape
    B = s0 * s1
    idx = x.reshape(B).astype(jnp.int32)
    out = _make_gather(B)(table, idx)
    return out.reshape(s0, s1, D_M)


# speedup vs baseline: 1.1195x; 1.1195x over previous
import functools

import jax
import jax.numpy as jnp
from jax import lax
from jax.experimental import pallas as pl
from jax.experimental.pallas import tpu as pltpu
from jax.experimental.pallas import tpu_sc as plsc

D_M = 64
SCALE = 8.0
NC = 2
NS = 16
NW = NC * NS
LANES = 16


def _make_gather(B):
    assert B % NW == 0
    bpw = B // NW
    C = 640
    assert bpw % (2 * C) == 0
    nchunk = bpw // C

    mesh = plsc.VectorSubcoreMesh(core_axis_name="c", subcore_axis_name="s")

    @functools.partial(
        pl.kernel,
        mesh=mesh,
        out_type=jax.ShapeDtypeStruct((B, D_M), jnp.float32),
        scratch_types=[
            pltpu.VMEM((bpw,), jnp.int32),
            pltpu.VMEM((C, D_M), jnp.float32),
            pltpu.VMEM((C, D_M), jnp.float32),
            pltpu.SemaphoreType.DMA,
            pltpu.SemaphoreType.DMA,
            pltpu.SemaphoreType.DMA,
            pltpu.SemaphoreType.DMA,
        ],
        compiler_params=pltpu.CompilerParams(use_tc_tiling_on_sc=False),
    )
    def k(table_hbm, idx_hbm, out_hbm, idx_v, rows0, rows1, gs0, gs1, ss0, ss1):
        wid = lax.axis_index("s") * NC + lax.axis_index("c")
        base = wid * bpw
        pltpu.sync_copy(idx_hbm.at[pl.ds(base, bpw)], idx_v)

        rows = (rows0, rows1)
        gsem = (gs0, gs1)
        ssem = (ss0, ss1)

        def start_gather(g, b):
            pltpu.async_copy(
                table_hbm.at[idx_v.at[pl.ds(g * C, C)]], rows[b], gsem[b])

        def wait_gather(b):
            pltpu.make_async_copy(
                table_hbm.at[idx_v.at[pl.ds(0, C)]], rows[b], gsem[b]).wait()

        def start_scatter(g, b):
            pltpu.async_copy(
                rows[b], out_hbm.at[pl.ds(base + g * C, C)], ssem[b])

        def wait_scatter(b):
            pltpu.make_async_copy(
                rows[b], out_hbm.at[pl.ds(base, C)], ssem[b]).wait()

        def scale(b):
            r = rows[b]
            UNROLL = 4

            def body(i, c):
                for u in range(UNROLL):
                    row = UNROLL * i + u
                    for j in range(D_M // LANES):
                        sl = pl.ds(j * LANES, LANES)
                        r[row, sl] = r[row, sl] * SCALE
                return c

            lax.fori_loop(0, C // UNROLL, body, 0)

        start_gather(0, 0)

        def outer(go, carry):
            for b in range(2):
                g = 2 * go + b
                ob = 1 - b
                wait_gather(b)

                @pl.when(g >= 1)
                def _():
                    wait_scatter(ob)

                @pl.when(g + 1 < nchunk)
                def _():
                    start_gather(g + 1, ob)

                scale(b)
                start_scatter(g, b)
            return carry

        lax.fori_loop(0, nchunk // 2, outer, 0)
        wait_scatter(1)

    return k


def kernel(x, table):
    s0, s1 = x.shape
    B = s0 * s1
    idx = x.reshape(B).astype(jnp.int32)
    out = _make_gather(B)(table, idx)
    return out.reshape(s0, s1, D_M)


# speedup vs baseline: 1.7923x; 1.6009x over previous
import functools

import jax
import jax.numpy as jnp
import numpy as np
from jax import lax
from jax.experimental import pallas as pl
from jax.experimental.pallas import tpu as pltpu
from jax.experimental.pallas import tpu_sc as plsc

D_M = 64
DP = 128
SCALE = 8.0
NC = 2
NS = 16
NW = NC * NS


def _transpose_scale(tableT):
    D, V = tableT.shape
    NCB = 4096
    grid = (pl.cdiv(V, NCB),)
    eye8 = jnp.asarray(SCALE * np.eye(D_M, dtype=np.float32))

    def body(x_ref, eye_ref, o_ref):
        y = jax.lax.dot_general(
            x_ref[...], eye_ref[...], (((0,), (0,)), ((), ())),
            preferred_element_type=jnp.float32)
        o_ref[:, : D_M] = y

    return pl.pallas_call(
        body,
        grid=grid,
        in_specs=[
            pl.BlockSpec((D, NCB), lambda i: (0, i)),
            pl.BlockSpec((D_M, D_M), lambda i: (0, 0)),
        ],
        out_specs=pl.BlockSpec((NCB, DP), lambda i: (i, 0)),
        out_shape=jax.ShapeDtypeStruct((V, DP), jnp.float32),
    )(tableT, eye8)


def _make_gather(B, V):
    assert B % NW == 0
    bpw = B // NW
    C = 400
    assert bpw % (2 * C) == 0
    nchunk = bpw // C

    mesh = plsc.VectorSubcoreMesh(core_axis_name="c", subcore_axis_name="s")

    @functools.partial(
        pl.kernel,
        mesh=mesh,
        out_type=jax.ShapeDtypeStruct((B, DP), jnp.float32),
        scratch_types=[
            pltpu.VMEM((bpw,), jnp.int32),
            pltpu.VMEM((C, DP), jnp.float32),
            pltpu.VMEM((C, DP), jnp.float32),
            pltpu.SemaphoreType.DMA,
            pltpu.SemaphoreType.DMA,
            pltpu.SemaphoreType.DMA,
            pltpu.SemaphoreType.DMA,
        ],
        compiler_params=pltpu.CompilerParams(use_tc_tiling_on_sc=False),
    )
    def k(table_hbm, idx_hbm, out_hbm, idx_v, rows0, rows1, gs0, gs1, ss0, ss1):
        wid = lax.axis_index("s") * NC + lax.axis_index("c")
        base = wid * bpw
        pltpu.sync_copy(idx_hbm.at[pl.ds(base, bpw)], idx_v)

        rows = (rows0, rows1)
        gsem = (gs0, gs1)
        ssem = (ss0, ss1)

        def start_gather(g, b):
            pltpu.async_copy(
                table_hbm.at[idx_v.at[pl.ds(g * C, C)]], rows[b], gsem[b])

        def wait_gather(b):
            pltpu.make_async_copy(
                table_hbm.at[idx_v.at[pl.ds(0, C)]], rows[b], gsem[b]).wait()

        def start_scatter(g, b):
            pltpu.async_copy(
                rows[b].at[pl.ds(0, C), pl.ds(0, D_M)],
                out_hbm.at[pl.ds(base + g * C, C), pl.ds(0, D_M)],
                ssem[b])

        def wait_scatter(b):
            pltpu.make_async_copy(
                rows[b].at[pl.ds(0, C), pl.ds(0, D_M)],
                out_hbm.at[pl.ds(base, C), pl.ds(0, D_M)],
                ssem[b]).wait()

        start_gather(0, 0)

        def outer(go, carry):
            for b in range(2):
                g = 2 * go + b
                ob = 1 - b
                wait_gather(b)

                @pl.when(g >= 1)
                def _():
                    wait_scatter(ob)

                @pl.when(g + 1 < nchunk)
                def _():
                    start_gather(g + 1, ob)

                start_scatter(g, b)
            return carry

        lax.fori_loop(0, nchunk // 2, outer, 0)
        wait_scatter(1)

    return k


def kernel(x, table):
    s0, s1 = x.shape
    B = s0 * s1
    V = table.shape[0]
    idx = x.reshape(B).astype(jnp.int32)
    tableP = _transpose_scale(table.T)
    out128 = _make_gather(B, V)(tableP, idx)
    return out128.reshape(s0, s1, DP)[:, :, :D_M]


# speedup vs baseline: 1.9864x; 1.1083x over previous
import functools

import jax
import jax.numpy as jnp
import numpy as np
from jax import lax
from jax.experimental import pallas as pl
from jax.experimental.pallas import tpu as pltpu
from jax.experimental.pallas import tpu_sc as plsc

D_M = 64
DP = 128
SCALE = 8.0
NC = 2
NS = 16
NW = NC * NS


def _transpose_scale(tableT):
    D, V = tableT.shape
    NCB = 8192
    grid = (pl.cdiv(V, NCB),)
    eye8 = jnp.asarray(
        SCALE * np.concatenate([np.eye(D_M, dtype=np.float32)] * 2, axis=1))

    def body(x_ref, eye_ref, o_ref):
        o_ref[...] = jax.lax.dot_general(
            x_ref[...], eye_ref[...], (((0,), (0,)), ((), ())),
            preferred_element_type=jnp.float32)

    return pl.pallas_call(
        body,
        grid=grid,
        in_specs=[
            pl.BlockSpec((D, NCB), lambda i: (0, i)),
            pl.BlockSpec((D_M, DP), lambda i: (0, 0)),
        ],
        out_specs=pl.BlockSpec((NCB, DP), lambda i: (i, 0)),
        out_shape=jax.ShapeDtypeStruct((V, DP), jnp.float32),
    )(tableT, eye8)


def _make_gather(B, V):
    assert B % NW == 0
    bpw = B // NW
    C = 400
    assert bpw % (2 * C) == 0
    nchunk = bpw // C

    mesh = plsc.VectorSubcoreMesh(core_axis_name="c", subcore_axis_name="s")

    @functools.partial(
        pl.kernel,
        mesh=mesh,
        out_type=jax.ShapeDtypeStruct((B, DP), jnp.float32),
        scratch_types=[
            pltpu.VMEM((bpw,), jnp.int32),
            pltpu.VMEM((C, DP), jnp.float32),
            pltpu.VMEM((C, DP), jnp.float32),
            pltpu.SemaphoreType.DMA,
            pltpu.SemaphoreType.DMA,
            pltpu.SemaphoreType.DMA,
            pltpu.SemaphoreType.DMA,
        ],
        compiler_params=pltpu.CompilerParams(use_tc_tiling_on_sc=False),
    )
    def k(table_hbm, idx_hbm, out_hbm, idx_v, rows0, rows1, gs0, gs1, ss0, ss1):
        wid = lax.axis_index("s") * NC + lax.axis_index("c")
        base = wid * bpw
        pltpu.sync_copy(idx_hbm.at[pl.ds(base, bpw)], idx_v)

        rows = (rows0, rows1)
        gsem = (gs0, gs1)
        ssem = (ss0, ss1)

        def start_gather(g, b):
            pltpu.async_copy(
                table_hbm.at[idx_v.at[pl.ds(g * C, C)]], rows[b], gsem[b])

        def wait_gather(b):
            pltpu.make_async_copy(
                table_hbm.at[idx_v.at[pl.ds(0, C)]], rows[b], gsem[b]).wait()

        def start_scatter(g, b):
            pltpu.async_copy(
                rows[b].at[pl.ds(0, C), pl.ds(0, D_M)],
                out_hbm.at[pl.ds(base + g * C, C), pl.ds(0, D_M)],
                ssem[b])

        def wait_scatter(b):
            pltpu.make_async_copy(
                rows[b].at[pl.ds(0, C), pl.ds(0, D_M)],
                out_hbm.at[pl.ds(base, C), pl.ds(0, D_M)],
                ssem[b]).wait()

        start_gather(0, 0)

        def outer(go, carry):
            for b in range(2):
                g = 2 * go + b
                ob = 1 - b
                wait_gather(b)

                @pl.when(g >= 1)
                def _():
                    wait_scatter(ob)

                @pl.when(g + 1 < nchunk)
                def _():
                    start_gather(g + 1, ob)

                start_scatter(g, b)
            return carry

        lax.fori_loop(0, nchunk // 2, outer, 0)
        wait_scatter(1)

    return k


def kernel(x, table):
    s0, s1 = x.shape
    B = s0 * s1
    V = table.shape[0]
    idx = x.reshape(B).astype(jnp.int32)
    tableP = _transpose_scale(table.T)
    out128 = _make_gather(B, V)(tableP, idx)
    return out128.reshape(s0, s1, DP)[:, :, :D_M]


# speedup vs baseline: 2.0588x; 1.0364x over previous
import functools

import jax
import jax.numpy as jnp
import numpy as np
from jax import lax
from jax.experimental import pallas as pl
from jax.experimental.pallas import tpu as pltpu
from jax.experimental.pallas import tpu_sc as plsc

D_M = 64
DP = 128
SCALE = 8.0
NC = 2
NS = 16
NW = NC * NS


def _transpose_scale(tableT):
    D, V = tableT.shape
    NCB = 16384
    grid = (pl.cdiv(V, NCB),)
    eye8 = jnp.asarray(
        SCALE * np.concatenate([np.eye(D_M, dtype=np.float32)] * 2, axis=1))

    def body(x_ref, eye_ref, o_ref):
        o_ref[...] = jax.lax.dot_general(
            x_ref[...], eye_ref[...], (((0,), (0,)), ((), ())),
            preferred_element_type=jnp.float32)

    return pl.pallas_call(
        body,
        grid=grid,
        in_specs=[
            pl.BlockSpec((D, NCB), lambda i: (0, i)),
            pl.BlockSpec((D_M, DP), lambda i: (0, 0)),
        ],
        out_specs=pl.BlockSpec((NCB, DP), lambda i: (i, 0)),
        out_shape=jax.ShapeDtypeStruct((V, DP), jnp.float32),
    )(tableT, eye8)


def _make_gather(B, V):
    assert B % NW == 0
    bpw = B // NW
    C = 400
    assert bpw % (2 * C) == 0
    nchunk = bpw // C

    mesh = plsc.VectorSubcoreMesh(core_axis_name="c", subcore_axis_name="s")

    @functools.partial(
        pl.kernel,
        mesh=mesh,
        out_type=jax.ShapeDtypeStruct((B, DP), jnp.float32),
        scratch_types=[
            pltpu.VMEM((bpw,), jnp.int32),
            pltpu.VMEM((C, DP), jnp.float32),
            pltpu.VMEM((C, DP), jnp.float32),
            pltpu.SemaphoreType.DMA,
            pltpu.SemaphoreType.DMA,
            pltpu.SemaphoreType.DMA,
            pltpu.SemaphoreType.DMA,
        ],
        compiler_params=pltpu.CompilerParams(use_tc_tiling_on_sc=False),
    )
    def k(table_hbm, idx_hbm, out_hbm, idx_v, rows0, rows1, gs0, gs1, ss0, ss1):
        wid = lax.axis_index("s") * NC + lax.axis_index("c")
        base = wid * bpw
        pltpu.sync_copy(idx_hbm.at[pl.ds(base, bpw)], idx_v)

        rows = (rows0, rows1)
        gsem = (gs0, gs1)
        ssem = (ss0, ss1)

        def start_gather(g, b):
            pltpu.async_copy(
                table_hbm.at[idx_v.at[pl.ds(g * C, C)]], rows[b], gsem[b])

        def wait_gather(b):
            pltpu.make_async_copy(
                table_hbm.at[idx_v.at[pl.ds(0, C)]], rows[b], gsem[b]).wait()

        def start_scatter(g, b):
            pltpu.async_copy(
                rows[b].at[pl.ds(0, C), pl.ds(0, D_M)],
                out_hbm.at[pl.ds(base + g * C, C), pl.ds(0, D_M)],
                ssem[b])

        def wait_scatter(b):
            pltpu.make_async_copy(
                rows[b].at[pl.ds(0, C), pl.ds(0, D_M)],
                out_hbm.at[pl.ds(base, C), pl.ds(0, D_M)],
                ssem[b]).wait()

        start_gather(0, 0)

        def outer(go, carry):
            for b in range(2):
                g = 2 * go + b
                ob = 1 - b
                wait_gather(b)

                @pl.when(g >= 1)
                def _():
                    wait_scatter(ob)

                @pl.when(g + 1 < nchunk)
                def _():
                    start_gather(g + 1, ob)

                start_scatter(g, b)
            return carry

        lax.fori_loop(0, nchunk // 2, outer, 0)
        wait_scatter(1)

    return k


def kernel(x, table):
    s0, s1 = x.shape
    B = s0 * s1
    V = table.shape[0]
    idx = x.reshape(B).astype(jnp.int32)
    tableP = _transpose_scale(table.T)
    out128 = _make_gather(B, V)(tableP, idx)
    return out128.reshape(s0, s1, DP)[:, :, :D_M]


# speedup vs baseline: 2.0737x; 1.0072x over previous
import functools

import jax
import jax.numpy as jnp
import numpy as np
from jax import lax
from jax.experimental import pallas as pl
from jax.experimental.pallas import tpu as pltpu
from jax.experimental.pallas import tpu_sc as plsc

D_M = 64
DP = 128
SCALE = 8.0
NC = 2
NS = 16
NW = NC * NS


def _transpose_scale(tableT):
    D, V = tableT.shape
    NCB = 32768
    grid = (pl.cdiv(V, NCB),)
    eye8 = jnp.asarray(
        SCALE * np.concatenate([np.eye(D_M, dtype=np.float32)] * 2, axis=1))

    def body(x_ref, eye_ref, o_ref):
        o_ref[...] = jax.lax.dot_general(
            x_ref[...], eye_ref[...], (((0,), (0,)), ((), ())),
            preferred_element_type=jnp.float32)

    return pl.pallas_call(
        body,
        grid=grid,
        in_specs=[
            pl.BlockSpec((D, NCB), lambda i: (0, i)),
            pl.BlockSpec((D_M, DP), lambda i: (0, 0)),
        ],
        out_specs=pl.BlockSpec((NCB, DP), lambda i: (i, 0)),
        out_shape=jax.ShapeDtypeStruct((V, DP), jnp.float32),
    )(tableT, eye8)


def _make_gather(B, V):
    assert B % NW == 0
    bpw = B // NW
    C = 400
    assert bpw % (2 * C) == 0
    nchunk = bpw // C

    mesh = plsc.VectorSubcoreMesh(core_axis_name="c", subcore_axis_name="s")

    @functools.partial(
        pl.kernel,
        mesh=mesh,
        out_type=jax.ShapeDtypeStruct((B, DP), jnp.float32),
        scratch_types=[
            pltpu.VMEM((bpw,), jnp.int32),
            pltpu.VMEM((C, DP), jnp.float32),
            pltpu.VMEM((C, DP), jnp.float32),
            pltpu.SemaphoreType.DMA,
            pltpu.SemaphoreType.DMA,
            pltpu.SemaphoreType.DMA,
            pltpu.SemaphoreType.DMA,
        ],
        compiler_params=pltpu.CompilerParams(use_tc_tiling_on_sc=False),
    )
    def k(table_hbm, idx_hbm, out_hbm, idx_v, rows0, rows1, gs0, gs1, ss0, ss1):
        wid = lax.axis_index("s") * NC + lax.axis_index("c")
        base = wid * bpw
        pltpu.sync_copy(idx_hbm.at[pl.ds(base, bpw)], idx_v)

        rows = (rows0, rows1)
        gsem = (gs0, gs1)
        ssem = (ss0, ss1)

        def start_gather(g, b):
            pltpu.async_copy(
                table_hbm.at[idx_v.at[pl.ds(g * C, C)]], rows[b], gsem[b])

        def wait_gather(b):
            pltpu.make_async_copy(
                table_hbm.at[idx_v.at[pl.ds(0, C)]], rows[b], gsem[b]).wait()

        def start_scatter(g, b):
            pltpu.async_copy(
                rows[b].at[pl.ds(0, C), pl.ds(0, D_M)],
                out_hbm.at[pl.ds(base + g * C, C), pl.ds(0, D_M)],
                ssem[b])

        def wait_scatter(b):
            pltpu.make_async_copy(
                rows[b].at[pl.ds(0, C), pl.ds(0, D_M)],
                out_hbm.at[pl.ds(base, C), pl.ds(0, D_M)],
                ssem[b]).wait()

        start_gather(0, 0)

        def outer(go, carry):
            for b in range(2):
                g = 2 * go + b
                ob = 1 - b
                wait_gather(b)

                @pl.when(g >= 1)
                def _():
                    wait_scatter(ob)

                @pl.when(g + 1 < nchunk)
                def _():
                    start_gather(g + 1, ob)

                start_scatter(g, b)
            return carry

        lax.fori_loop(0, nchunk // 2, outer, 0)
        wait_scatter(1)

    return k


def kernel(x, table):
    s0, s1 = x.shape
    B = s0 * s1
    V = table.shape[0]
    idx = x.reshape(B).astype(jnp.int32)
    tableP = _transpose_scale(table.T)
    out128 = _make_gather(B, V)(tableP, idx)
    return out128.reshape(s0, s1, DP)[:, :, :D_M]


# speedup vs baseline: 2.3314x; 1.1243x over previous
import functools

import jax
import jax.numpy as jnp
import numpy as np
from jax import lax
from jax.experimental import pallas as pl
from jax.experimental.pallas import tpu as pltpu
from jax.experimental.pallas import tpu_sc as plsc

D_M = 64
DP = 128
SCALE = 8.0
NC = 2
NS = 16
NW = NC * NS


def _transpose_scale(tableT):
    D, V = tableT.shape
    NCB = 16384
    grid = (pl.cdiv(V, NCB),)
    eye8 = jnp.asarray(
        SCALE * np.concatenate([np.eye(D_M, dtype=np.float32)] * 2, axis=1))

    def body(x_ref, eye_ref, o_ref):
        o_ref[...] = jax.lax.dot_general(
            x_ref[...], eye_ref[...], (((0,), (0,)), ((), ())),
            preferred_element_type=jnp.float32)

    return pl.pallas_call(
        body,
        grid=grid,
        in_specs=[
            pl.BlockSpec((D, NCB), lambda i: (0, i)),
            pl.BlockSpec((D_M, DP), lambda i: (0, 0)),
        ],
        out_specs=pl.BlockSpec((NCB, DP), lambda i: (i, 0)),
        out_shape=jax.ShapeDtypeStruct((V, DP), jnp.float32),
    )(tableT, eye8)


def _make_gather(B, V):
    assert B % NW == 0
    bpw = B // NW
    C = 800
    assert bpw % (2 * C) == 0
    nchunk = bpw // C

    mesh = plsc.VectorSubcoreMesh(core_axis_name="c", subcore_axis_name="s")

    @functools.partial(
        pl.kernel,
        mesh=mesh,
        out_type=jax.ShapeDtypeStruct((B, DP), jnp.float32),
        scratch_types=[
            pltpu.VMEM((bpw,), jnp.int32),
            pltpu.VMEM((C, D_M), jnp.float32),
            pltpu.VMEM((C, D_M), jnp.float32),
            pltpu.SemaphoreType.DMA,
            pltpu.SemaphoreType.DMA,
            pltpu.SemaphoreType.DMA,
            pltpu.SemaphoreType.DMA,
        ],
        compiler_params=pltpu.CompilerParams(use_tc_tiling_on_sc=False),
    )
    def k(table_hbm, idx_hbm, out_hbm, idx_v, rows0, rows1, gs0, gs1, ss0, ss1):
        wid = lax.axis_index("s") * NC + lax.axis_index("c")
        base = wid * bpw
        pltpu.sync_copy(idx_hbm.at[pl.ds(base, bpw)], idx_v)

        rows = (rows0, rows1)
        gsem = (gs0, gs1)
        ssem = (ss0, ss1)

        def start_gather(g, b):
            pltpu.async_copy(
                table_hbm.at[idx_v.at[pl.ds(g * C, C)]], rows[b], gsem[b])

        def wait_gather(b):
            pltpu.make_async_copy(
                table_hbm.at[idx_v.at[pl.ds(0, C)]], rows[b], gsem[b]).wait()

        def start_scatter(g, b):
            pltpu.async_copy(
                rows[b],
                out_hbm.at[pl.ds(base + g * C, C), pl.ds(0, D_M)],
                ssem[b])

        def wait_scatter(b):
            pltpu.make_async_copy(
                rows[b],
                out_hbm.at[pl.ds(base, C), pl.ds(0, D_M)],
                ssem[b]).wait()

        start_gather(0, 0)

        def outer(go, carry):
            for b in range(2):
                g = 2 * go + b
                ob = 1 - b
                wait_gather(b)

                @pl.when(g >= 1)
                def _():
                    wait_scatter(ob)

                @pl.when(g + 1 < nchunk)
                def _():
                    start_gather(g + 1, ob)

                start_scatter(g, b)
            return carry

        lax.fori_loop(0, nchunk // 2, outer, 0)
        wait_scatter(1)

    return k


def kernel(x, table):
    s0, s1 = x.shape
    B = s0 * s1
    V = table.shape[0]
    idx2 = x.reshape(B).astype(jnp.int32) << 1
    tableP = _transpose_scale(table.T)
    table64 = tableP.reshape(2 * V, D_M)
    out128 = _make_gather(B, V)(table64, idx2)
    return out128.reshape(s0, s1, DP)[:, :, :D_M]
